# baseline (speedup 1.0000x reference)
import functools

import jax
import jax.numpy as jnp
from jax import lax
from jax.experimental import pallas as pl
from jax.experimental.pallas import tpu as pltpu
from jax.experimental.pallas import tpu_sc as plsc

D_MODEL = 64
MAX_SEQ_LEN = 256

_INFO = plsc.get_sparse_core_info()
_NC = _INFO.num_cores
_NS = _INFO.num_subcores
_L = _INFO.num_lanes
_NW = _NC * _NS


def _pe_row(pos):
    j = jnp.arange(D_MODEL, dtype=jnp.float32)
    angle = pos / jnp.power(10000.0, 2.0 * j / D_MODEL)
    even = (jnp.arange(D_MODEL) % 2 == 0)
    return jnp.where(even, jnp.sin(angle), jnp.cos(angle))


@functools.lru_cache(maxsize=None)
def _make_kernel(B, V):
    assert B % _NW == 0
    b_per_w = B // _NW
    C = 1600
    assert b_per_w % C == 0
    n_chunks = b_per_w // C
    n_sub = D_MODEL // _L

    mesh = plsc.VectorSubcoreMesh(core_axis_name="c", subcore_axis_name="s")

    @functools.partial(
        pl.kernel,
        mesh=mesh,
        compiler_params=pltpu.CompilerParams(use_tc_tiling_on_sc=False),
        out_type=jax.ShapeDtypeStruct((B, D_MODEL), jnp.float32),
        scratch_types=[
            pltpu.VMEM((b_per_w,), jnp.int32),
            pltpu.VMEM((C, D_MODEL), jnp.float32),
            pltpu.VMEM((D_MODEL,), jnp.float32),
            pltpu.SemaphoreType.DMA,
        ],
    )
    def body(idx_hbm, table_hbm, pe_hbm, out_hbm, idx_v, rows_v, pe_v, sem):
        wid = lax.axis_index("s") * _NC + lax.axis_index("c")
        base = wid * b_per_w
        pltpu.sync_copy(pe_hbm, pe_v)
        pltpu.sync_copy(idx_hbm.at[pl.ds(base, b_per_w)], idx_v)
        pe_regs = [pe_v[pl.ds(k * _L, _L)] for k in range(n_sub)]
        for g in range(n_chunks):
            pltpu.async_copy(
                table_hbm.at[idx_v.at[pl.ds(g * C, C)]], rows_v, sem
            ).wait()

            def add_pe(r, _):
                for k in range(n_sub):
                    rows_v[r, pl.ds(k * _L, _L)] += pe_regs[k]
                return 0

            lax.fori_loop(0, C, add_pe, 0, unroll=4)
            pltpu.sync_copy(rows_v, out_hbm.at[pl.ds(base + g * C, C)])

    return body


def kernel(x, table):
    Bb, Ls = x.shape
    V, D = table.shape
    pe = _pe_row(Ls)
    out = _make_kernel(Bb * Ls, V)(x.reshape(-1), table, pe)
    return out.reshape(Bb, Ls, D)
